# Initial kernel scaffold; baseline (speedup 1.0000x reference)
import jax
import jax.numpy as jnp
from jax.experimental import pallas as pl


def kernel(x, t, c):
    raise NotImplementedError("write your pallas kernel here")


# speedup vs baseline: 580.7464x; 580.7464x over previous
import functools

import jax
import jax.numpy as jnp
from jax import lax
from jax.experimental import pallas as pl
from jax.experimental.pallas import tpu as pltpu
from jax.experimental.pallas import tpu_sc as plsc

N = 4194304
T_DIM = 4096
L = 16
NW = 32
PER_W = N // NW
CHUNK = 8192
N_CHUNKS = PER_W // CHUNK
SEARCH_STEPS = 12


def _deboor_vec(xv, t_v, c_v):
    lo = jnp.zeros((L,), jnp.int32)
    hi = jnp.full((L,), T_DIM, jnp.int32)
    for _ in range(SEARCH_STEPS):
        mid = (lo + hi) >> 1
        tm = plsc.load_gather(t_v, [mid])
        pred = tm <= xv
        lo = jnp.where(pred, mid + 1, lo)
        hi = jnp.where(pred, hi, mid)
    k = jnp.clip(lo - 1, 3, T_DIM - 5)

    c0 = plsc.load_gather(c_v, [k - 3])
    c1 = plsc.load_gather(c_v, [k - 2])
    c2 = plsc.load_gather(c_v, [k - 1])
    c3 = plsc.load_gather(c_v, [k])
    tm2 = plsc.load_gather(t_v, [k - 2])
    tm1 = plsc.load_gather(t_v, [k - 1])
    t0 = plsc.load_gather(t_v, [k])
    t1 = plsc.load_gather(t_v, [k + 1])
    t2 = plsc.load_gather(t_v, [k + 2])
    t3 = plsc.load_gather(t_v, [k + 3])

    one = jnp.float32(1.0)
    a3 = (xv - t0) / (t3 - t0)
    d3 = (one - a3) * c2 + a3 * c3
    a2 = (xv - tm1) / (t2 - tm1)
    d2 = (one - a2) * c1 + a2 * c2
    a1 = (xv - tm2) / (t1 - tm2)
    d1 = (one - a1) * c0 + a1 * c1
    b3 = (xv - t0) / (t2 - t0)
    e3 = (one - b3) * d2 + b3 * d3
    b2 = (xv - tm1) / (t1 - tm1)
    e2 = (one - b2) * d1 + b2 * d2
    g3 = (xv - t0) / (t1 - t0)
    return (one - g3) * e2 + g3 * e3


def kernel(x, t, c):
    mesh = plsc.VectorSubcoreMesh(core_axis_name="c", subcore_axis_name="s")

    @functools.partial(
        pl.kernel,
        mesh=mesh,
        out_type=jax.ShapeDtypeStruct((N,), jnp.float32),
        compiler_params=pltpu.CompilerParams(needs_layout_passes=False),
        scratch_types=[
            pltpu.VMEM((T_DIM,), jnp.float32),
            pltpu.VMEM((T_DIM,), jnp.float32),
            pltpu.VMEM((CHUNK,), jnp.float32),
            pltpu.VMEM((CHUNK,), jnp.float32),
        ],
    )
    def run(x_hbm, t_hbm, c_hbm, o_hbm, t_v, c_v, x_v, o_v):
        wid = lax.axis_index("s") * 2 + lax.axis_index("c")
        base = wid * PER_W
        pltpu.sync_copy(t_hbm, t_v)
        pltpu.sync_copy(c_hbm, c_v)

        def chunk_body(ci, carry):
            off = base + ci * CHUNK
            pltpu.sync_copy(x_hbm.at[pl.ds(off, CHUNK)], x_v)

            def vec_body(i, inner):
                xv = x_v[pl.ds(i * L, L)]
                o_v[pl.ds(i * L, L)] = _deboor_vec(xv, t_v, c_v)
                return inner

            lax.fori_loop(0, CHUNK // L, vec_body, 0)
            pltpu.sync_copy(o_v, o_hbm.at[pl.ds(off, CHUNK)])
            return carry

        lax.fori_loop(0, N_CHUNKS, chunk_body, 0)

    return run(x, t, c)


# speedup vs baseline: 1377.2796x; 2.3716x over previous
import functools

import jax
import jax.numpy as jnp
from jax import lax
from jax.experimental import pallas as pl
from jax.experimental.pallas import tpu as pltpu
from jax.experimental.pallas import tpu_sc as plsc

N = 4194304
T_DIM = 4096
L = 16
NW = 32
PER_W = N // NW
CHUNK = 32768
N_CHUNKS = PER_W // CHUNK
U = 4
STEPS = (2048, 1024, 512, 256, 128, 64, 32, 16, 8, 4, 2, 1)


def _deboor_block(i, x_v, o_v, t_v, c_v):
    xs = [x_v[pl.ds((i * U + u) * L, L)] for u in range(U)]
    los = [jnp.zeros((L,), jnp.int32) for _ in range(U)]
    for step in STEPS:
        idxs = [los[u] + (step - 1) for u in range(U)]
        tms = [plsc.load_gather(t_v, [idxs[u]]) for u in range(U)]
        los = [
            jnp.where(tms[u] <= xs[u], idxs[u] + 1, los[u])
            for u in range(U)
        ]
    ks = [
        jnp.minimum(jnp.maximum(los[u] - 1, 3), T_DIM - 5) for u in range(U)
    ]

    for u in range(U):
        k = ks[u]
        xv = xs[u]
        c0 = plsc.load_gather(c_v, [k - 3])
        c1 = plsc.load_gather(c_v, [k - 2])
        c2 = plsc.load_gather(c_v, [k - 1])
        c3 = plsc.load_gather(c_v, [k])
        tm2 = plsc.load_gather(t_v, [k - 2])
        tm1 = plsc.load_gather(t_v, [k - 1])
        t0 = plsc.load_gather(t_v, [k])
        t1 = plsc.load_gather(t_v, [k + 1])
        t2 = plsc.load_gather(t_v, [k + 2])
        t3 = plsc.load_gather(t_v, [k + 3])

        one = jnp.float32(1.0)
        a3 = (xv - t0) / (t3 - t0)
        d3 = (one - a3) * c2 + a3 * c3
        a2 = (xv - tm1) / (t2 - tm1)
        d2 = (one - a2) * c1 + a2 * c2
        a1 = (xv - tm2) / (t1 - tm2)
        d1 = (one - a1) * c0 + a1 * c1
        b3 = (xv - t0) / (t2 - t0)
        e3 = (one - b3) * d2 + b3 * d3
        b2 = (xv - tm1) / (t1 - tm1)
        e2 = (one - b2) * d1 + b2 * d2
        g3 = (xv - t0) / (t1 - t0)
        o_v[pl.ds((i * U + u) * L, L)] = (one - g3) * e2 + g3 * e3


def kernel(x, t, c):
    mesh = plsc.VectorSubcoreMesh(core_axis_name="c", subcore_axis_name="s")

    @functools.partial(
        pl.kernel,
        mesh=mesh,
        out_type=jax.ShapeDtypeStruct((N,), jnp.float32),
        compiler_params=pltpu.CompilerParams(needs_layout_passes=False),
        scratch_types=[
            pltpu.VMEM((T_DIM,), jnp.float32),
            pltpu.VMEM((T_DIM,), jnp.float32),
            pltpu.VMEM((CHUNK,), jnp.float32),
            pltpu.VMEM((CHUNK,), jnp.float32),
        ],
    )
    def run(x_hbm, t_hbm, c_hbm, o_hbm, t_v, c_v, x_v, o_v):
        wid = lax.axis_index("s") * 2 + lax.axis_index("c")
        base = wid * PER_W
        pltpu.sync_copy(t_hbm, t_v)
        pltpu.sync_copy(c_hbm, c_v)

        def chunk_body(ci, carry):
            off = base + ci * CHUNK
            pltpu.sync_copy(x_hbm.at[pl.ds(off, CHUNK)], x_v)

            @plsc.parallel_loop(0, CHUNK // (L * U))
            def _(i):
                _deboor_block(i, x_v, o_v, t_v, c_v)

            pltpu.sync_copy(o_v, o_hbm.at[pl.ds(off, CHUNK)])
            return carry

        lax.fori_loop(0, N_CHUNKS, chunk_body, 0)

    return run(x, t, c)
